# Initial kernel scaffold; baseline (speedup 1.0000x reference)
import jax
import jax.numpy as jnp
from jax.experimental import pallas as pl


def kernel(x, table):
    raise NotImplementedError("write your pallas kernel here")


# speedup vs baseline: 3.6156x; 3.6156x over previous
import functools

import jax
import jax.numpy as jnp
from jax import lax
from jax.experimental import pallas as pl
from jax.experimental.pallas import tpu as pltpu
from jax.experimental.pallas import tpu_sc as plsc

N_CLUSTERS = 512
DIM = 64
NC = 2
NS = 16
NW = NC * NS


@functools.lru_cache(maxsize=None)
def _embed_lookup(B: int, C: int):
    assert B % (NW * C) == 0
    b_per_w = B // NW
    n_chunks = b_per_w // C
    mesh = plsc.VectorSubcoreMesh(core_axis_name="c", subcore_axis_name="s")

    @functools.partial(
        pl.kernel,
        mesh=mesh,
        out_type=jax.ShapeDtypeStruct((B, DIM), jnp.float32),
        scratch_types=[
            pltpu.VMEM((C,), jnp.int32),
            pltpu.VMEM((C, DIM), jnp.float32),
            pltpu.SemaphoreType.DMA,
        ],
        compiler_params=pltpu.CompilerParams(use_tc_tiling_on_sc=False),
    )
    def k(idx_hbm, table_hbm, out_hbm, idx_v, rows_v, sem):
        wid = lax.axis_index("s") * NC + lax.axis_index("c")
        base = wid * b_per_w

        def body(i, carry):
            off = base + i * C
            pltpu.sync_copy(idx_hbm.at[pl.ds(off, C)], idx_v)
            pltpu.async_copy(table_hbm.at[idx_v], rows_v, sem).wait()
            pltpu.sync_copy(rows_v, out_hbm.at[pl.ds(off, C)])
            return carry

        lax.fori_loop(0, n_chunks, body, 0)

    return k


def kernel(x, table):
    B = x.shape[0] * x.shape[1]
    flat = x.reshape(B)
    out = _embed_lookup(B, 512)(flat, table)
    return out.reshape(x.shape[0], x.shape[1], DIM)


# speedup vs baseline: 5.8067x; 1.6060x over previous
import functools

import jax
import jax.numpy as jnp
from jax import lax
from jax.experimental import pallas as pl
from jax.experimental.pallas import tpu as pltpu
from jax.experimental.pallas import tpu_sc as plsc

N_CLUSTERS = 512
DIM = 64
NC = 2
NS = 16
NW = NC * NS
NBUF = 2


@functools.lru_cache(maxsize=None)
def _embed_lookup(B: int, C: int):
    assert B % (NW * C) == 0
    b_per_w = B // NW
    n_chunks = b_per_w // C
    assert n_chunks % NBUF == 0
    mesh = plsc.VectorSubcoreMesh(core_axis_name="c", subcore_axis_name="s")

    @functools.partial(
        pl.kernel,
        mesh=mesh,
        out_type=jax.ShapeDtypeStruct((B, DIM), jnp.float32),
        scratch_types=[
            pltpu.VMEM_SHARED((N_CLUSTERS, DIM), jnp.float32),
            pltpu.VMEM((N_CLUSTERS, DIM), jnp.float32),
            pltpu.VMEM((NBUF, C), jnp.int32),
            pltpu.VMEM((NBUF, C, DIM), jnp.float32),
            pltpu.SemaphoreType.DMA,
            pltpu.SemaphoreType.DMA,
            pltpu.SemaphoreType.DMA,
            pltpu.SemaphoreType.DMA,
            pltpu.SemaphoreType.DMA,
        ],
        compiler_params=pltpu.CompilerParams(use_tc_tiling_on_sc=False),
    )
    def k(idx_hbm, table_hbm, out_hbm, table_sh, bounce_v, idx_v, rows_v,
          isem0, isem1, osem0, osem1, gsem):
        isems = [isem0, isem1]
        osems = [osem0, osem1]
        sid = lax.axis_index("s")
        wid = sid * NC + lax.axis_index("c")
        base = wid * b_per_w

        @pl.when(sid == 0)
        def _():
            pltpu.sync_copy(table_hbm, bounce_v)
            pltpu.sync_copy(bounce_v, table_sh)

        plsc.subcore_barrier()

        for b in range(NBUF):
            pltpu.async_copy(
                idx_hbm.at[pl.ds(base + b * C, C)], idx_v.at[b], isems[b])

        def outer(i0, carry):
            for b in range(NBUF):
                i = i0 + b
                off = base + i * C

                @pl.when(i >= NBUF)
                def _():
                    pltpu.make_async_copy(
                        rows_v.at[b], out_hbm.at[pl.ds(base, C)],
                        osems[b]).wait()

                pltpu.make_async_copy(
                    idx_hbm.at[pl.ds(base, C)], idx_v.at[b], isems[b]).wait()

                pltpu.async_copy(
                    table_sh.at[idx_v.at[b]], rows_v.at[b], gsem).wait()

                pltpu.async_copy(
                    rows_v.at[b], out_hbm.at[pl.ds(off, C)], osems[b])

                @pl.when(i + NBUF < n_chunks)
                def _():
                    pltpu.async_copy(
                        idx_hbm.at[pl.ds(off + NBUF * C, C)], idx_v.at[b],
                        isems[b])
            return carry

        lax.fori_loop(0, n_chunks // NBUF, lambda j, c: outer(j * NBUF, c), 0)

        for b in range(NBUF):
            pltpu.make_async_copy(
                rows_v.at[b], out_hbm.at[pl.ds(base, C)], osems[b]).wait()

    return k


def kernel(x, table):
    B = x.shape[0] * x.shape[1]
    flat = x.reshape(B)
    out = _embed_lookup(B, 512)(flat, table)
    return out.reshape(x.shape[0], x.shape[1], DIM)


# speedup vs baseline: 5.8213x; 1.0025x over previous
import functools

import jax
import jax.numpy as jnp
from jax import lax
from jax.experimental import pallas as pl
from jax.experimental.pallas import tpu as pltpu
from jax.experimental.pallas import tpu_sc as plsc

N_CLUSTERS = 512
DIM = 64
NC = 2
NS = 16
NW = NC * NS
NBUF = 2


@functools.lru_cache(maxsize=None)
def _embed_lookup(NROW: int, NCOL: int, R: int):
    assert NROW % (NW * R) == 0
    rows_per_w = NROW // NW
    n_chunks = rows_per_w // R
    C = R * NCOL
    assert C >= N_CLUSTERS

    mesh = plsc.VectorSubcoreMesh(core_axis_name="c", subcore_axis_name="s")

    @functools.partial(
        pl.kernel,
        mesh=mesh,
        out_type=jax.ShapeDtypeStruct((NROW, NCOL, DIM), jnp.float32),
        scratch_types=[
            pltpu.VMEM_SHARED((N_CLUSTERS, DIM), jnp.float32),
            pltpu.VMEM((NBUF, C), jnp.int32),
            pltpu.VMEM((NBUF, C, DIM), jnp.float32),
            pltpu.SemaphoreType.DMA,
            pltpu.SemaphoreType.DMA,
            pltpu.SemaphoreType.DMA,
            pltpu.SemaphoreType.DMA,
            pltpu.SemaphoreType.DMA,
        ],
        compiler_params=pltpu.CompilerParams(use_tc_tiling_on_sc=False),
    )
    def k(flat_hbm, table_hbm, out_hbm, table_sh, idx_v, rows_v,
          isem0, isem1, osem0, osem1, gsem):
        isems = [isem0, isem1]
        osems = [osem0, osem1]
        sid = lax.axis_index("s")
        wid = sid * NC + lax.axis_index("c")
        rbase = wid * rows_per_w
        fbase = rbase * NCOL

        @pl.when(sid == 0)
        def _():
            bounce = rows_v.at[0].at[pl.ds(0, N_CLUSTERS)]
            pltpu.sync_copy(table_hbm, bounce)
            pltpu.sync_copy(bounce, table_sh)

        plsc.subcore_barrier()

        for b in range(NBUF):
            pltpu.async_copy(
                flat_hbm.at[pl.ds(fbase + b * C, C)], idx_v.at[b], isems[b])

        def step(i, b):
            @pl.when(i >= NBUF)
            def _():
                for _ in range(R):
                    pltpu.make_async_copy(
                        rows_v.at[b].at[pl.ds(0, NCOL)], out_hbm.at[rbase],
                        osems[b]).wait()

            pltpu.make_async_copy(
                flat_hbm.at[pl.ds(fbase, C)], idx_v.at[b], isems[b]).wait()

            pltpu.async_copy(
                table_sh.at[idx_v.at[b]], rows_v.at[b], gsem).wait()

            for r in range(R):
                pltpu.async_copy(
                    rows_v.at[b].at[pl.ds(r * NCOL, NCOL)],
                    out_hbm.at[rbase + i * R + r], osems[b])

            @pl.when(i + NBUF < n_chunks)
            def _():
                pltpu.async_copy(
                    flat_hbm.at[pl.ds(fbase + (i + NBUF) * C, C)],
                    idx_v.at[b], isems[b])

        def outer(j, carry):
            for b in range(NBUF):
                step(j * NBUF + b, b)
            return carry

        lax.fori_loop(0, n_chunks // NBUF, outer, 0)

        for b in range(NBUF):
            for _ in range(R):
                pltpu.make_async_copy(
                    rows_v.at[b].at[pl.ds(0, NCOL)], out_hbm.at[rbase],
                    osems[b]).wait()

    return k


def kernel(x, table):
    flat = x.reshape(x.shape[0] * x.shape[1])
    return _embed_lookup(x.shape[0], x.shape[1], 4)(flat, table)


# speedup vs baseline: 6.8292x; 1.1731x over previous
import functools

import jax
import jax.numpy as jnp
from jax import lax
from jax.experimental import pallas as pl
from jax.experimental.pallas import tpu as pltpu
from jax.experimental.pallas import tpu_sc as plsc

N_CLUSTERS = 512
DIM = 64
NC = 2
NS = 16
NW = NC * NS
UNROLL = 8


@functools.lru_cache(maxsize=None)
def _embed_lookup(NROW: int, NCOL: int):
    assert NROW % (NW * 2) == 0 and NCOL % UNROLL == 0
    rows_per_w = NROW // NW
    n_chunks = rows_per_w
    C = NCOL

    mesh = plsc.VectorSubcoreMesh(core_axis_name="c", subcore_axis_name="s")

    @functools.partial(
        pl.kernel,
        mesh=mesh,
        out_type=jax.ShapeDtypeStruct((NROW, NCOL, DIM), jnp.float32),
        scratch_types=[
            pltpu.VMEM_SHARED((N_CLUSTERS, 2 * DIM), jnp.float32),
            pltpu.VMEM((C,), jnp.int32),
            pltpu.VMEM((C,), jnp.int32),
            pltpu.VMEM((C, 2 * DIM), jnp.float32),
            pltpu.VMEM((C, 2 * DIM), jnp.float32),
            pltpu.VMEM((C, DIM), jnp.float32),
            pltpu.VMEM((C, DIM), jnp.float32),
            pltpu.SemaphoreType.DMA,
            pltpu.SemaphoreType.DMA,
            pltpu.SemaphoreType.DMA,
            pltpu.SemaphoreType.DMA,
            pltpu.SemaphoreType.DMA,
        ],
    )
    def k(flat_hbm, table2_hbm, out_hbm, table_sh,
          idx_v0, idx_v1, wide_v0, wide_v1, rows_v0, rows_v1,
          isem0, isem1, osem0, osem1, gsem):
        idx_vs = [idx_v0, idx_v1]
        wide_vs = [wide_v0, wide_v1]
        rows_vs = [rows_v0, rows_v1]
        isems = [isem0, isem1]
        osems = [osem0, osem1]
        sid = lax.axis_index("s")
        wid = sid * NC + lax.axis_index("c")
        rbase = wid * rows_per_w
        fbase = rbase * NCOL

        @pl.when(sid == 0)
        def _():
            for p in range(0, N_CLUSTERS, C):
                n = min(C, N_CLUSTERS - p)
                bounce = wide_v0.at[pl.ds(0, n)]
                pltpu.sync_copy(table2_hbm.at[pl.ds(p, n)], bounce)
                pltpu.sync_copy(bounce, table_sh.at[pl.ds(p, n)])

        plsc.subcore_barrier()

        for b in range(2):
            pltpu.async_copy(
                flat_hbm.at[pl.ds(fbase + b * C, C)], idx_vs[b], isems[b])

        def step(i, b):
            @pl.when(i >= 2)
            def _():
                pltpu.make_async_copy(
                    rows_vs[b], out_hbm.at[rbase], osems[b]).wait()

            pltpu.make_async_copy(
                flat_hbm.at[pl.ds(fbase, C)], idx_vs[b], isems[b]).wait()

            pltpu.async_copy(
                table_sh.at[idx_vs[b]], wide_vs[b], gsem).wait()

            @pl.when(i + 2 < n_chunks)
            def _():
                pltpu.async_copy(
                    flat_hbm.at[pl.ds(fbase + (i + 2) * C, C)],
                    idx_vs[b], isems[b])

            def repack(q, carry):
                for u in range(UNROLL):
                    r = q * UNROLL + u
                    for c in range(DIM // 16):
                        rows_vs[b][r, pl.ds(c * 16, 16)] = (
                            wide_vs[b][r, pl.ds(c * 16, 16)])
                return carry

            lax.fori_loop(0, C // UNROLL, repack, 0)

            pltpu.async_copy(rows_vs[b], out_hbm.at[rbase + i], osems[b])

        def outer(j, carry):
            step(2 * j, 0)
            step(2 * j + 1, 1)
            return carry

        lax.fori_loop(0, n_chunks // 2, outer, 0)

        for b in range(2):
            pltpu.make_async_copy(
                rows_vs[b], out_hbm.at[rbase], osems[b]).wait()

    return k


def kernel(x, table):
    flat = x.reshape(x.shape[0] * x.shape[1])
    table2 = jnp.concatenate([table, table], axis=1)
    return _embed_lookup(x.shape[0], x.shape[1])(flat, table2)


# speedup vs baseline: 7.0653x; 1.0346x over previous
import functools

import jax
import jax.numpy as jnp
from jax import lax
from jax.experimental import pallas as pl
from jax.experimental.pallas import tpu as pltpu
from jax.experimental.pallas import tpu_sc as plsc

N_CLUSTERS = 512
DIM = 64
NC = 2
NS = 16
NW = NC * NS
UNROLL = 8


@functools.lru_cache(maxsize=None)
def _embed_lookup(NROW: int, NCOL: int):
    assert NROW % (NW * 2) == 0 and NCOL % UNROLL == 0
    rows_per_w = NROW // NW
    n_chunks = rows_per_w
    C = NCOL

    mesh = plsc.VectorSubcoreMesh(core_axis_name="c", subcore_axis_name="s")

    @functools.partial(
        pl.kernel,
        mesh=mesh,
        out_type=jax.ShapeDtypeStruct((NROW, NCOL, DIM), jnp.float32),
        scratch_types=[
            pltpu.VMEM_SHARED((N_CLUSTERS, 2 * DIM), jnp.float32),
            pltpu.VMEM((C,), jnp.int32),
            pltpu.VMEM((C,), jnp.int32),
            pltpu.VMEM((C, 2 * DIM), jnp.float32),
            pltpu.VMEM((C, 2 * DIM), jnp.float32),
            pltpu.VMEM((C, DIM), jnp.float32),
            pltpu.VMEM((C, DIM), jnp.float32),
            pltpu.SemaphoreType.DMA,
            pltpu.SemaphoreType.DMA,
            pltpu.SemaphoreType.DMA,
            pltpu.SemaphoreType.DMA,
            pltpu.SemaphoreType.DMA,
            pltpu.SemaphoreType.DMA,
        ],
    )
    def k(flat_hbm, table2_hbm, out_hbm, table_sh,
          idx_v0, idx_v1, wide_v0, wide_v1, rows_v0, rows_v1,
          isem0, isem1, osem0, osem1, gsem0, gsem1):
        idx_vs = [idx_v0, idx_v1]
        wide_vs = [wide_v0, wide_v1]
        rows_vs = [rows_v0, rows_v1]
        isems = [isem0, isem1]
        osems = [osem0, osem1]
        gsems = [gsem0, gsem1]
        sid = lax.axis_index("s")
        wid = sid * NC + lax.axis_index("c")
        rbase = wid * rows_per_w
        fbase = rbase * NCOL

        @pl.when(sid == 0)
        def _():
            for p in range(0, N_CLUSTERS, C):
                n = min(C, N_CLUSTERS - p)
                bounce = wide_v0.at[pl.ds(0, n)]
                pltpu.sync_copy(table2_hbm.at[pl.ds(p, n)], bounce)
                pltpu.sync_copy(bounce, table_sh.at[pl.ds(p, n)])

        plsc.subcore_barrier()

        for b in range(2):
            pltpu.async_copy(
                flat_hbm.at[pl.ds(fbase + b * C, C)], idx_vs[b], isems[b])
        pltpu.make_async_copy(
            flat_hbm.at[pl.ds(fbase, C)], idx_vs[0], isems[0]).wait()
        pltpu.async_copy(table_sh.at[idx_vs[0]], wide_vs[0], gsems[0])

        def step(i, b):
            b1 = 1 - b

            @pl.when(i >= 2)
            def _():
                pltpu.make_async_copy(
                    rows_vs[b], out_hbm.at[rbase], osems[b]).wait()

            pltpu.make_async_copy(
                table_sh.at[idx_vs[b]], wide_vs[b], gsems[b]).wait()

            @pl.when(i + 2 < n_chunks)
            def _():
                pltpu.async_copy(
                    flat_hbm.at[pl.ds(fbase + (i + 2) * C, C)],
                    idx_vs[b], isems[b])

            @pl.when(i + 1 < n_chunks)
            def _():
                pltpu.make_async_copy(
                    flat_hbm.at[pl.ds(fbase, C)], idx_vs[b1],
                    isems[b1]).wait()
                pltpu.async_copy(
                    table_sh.at[idx_vs[b1]], wide_vs[b1], gsems[b1])

            def repack(q, carry):
                for u in range(UNROLL):
                    r = q * UNROLL + u
                    for c in range(DIM // 16):
                        rows_vs[b][r, pl.ds(c * 16, 16)] = (
                            wide_vs[b][r, pl.ds(c * 16, 16)])
                return carry

            lax.fori_loop(0, C // UNROLL, repack, 0)

            pltpu.async_copy(rows_vs[b], out_hbm.at[rbase + i], osems[b])

        def outer(j, carry):
            step(2 * j, 0)
            step(2 * j + 1, 1)
            return carry

        lax.fori_loop(0, n_chunks // 2, outer, 0)

        for b in range(2):
            pltpu.make_async_copy(
                rows_vs[b], out_hbm.at[rbase], osems[b]).wait()

    return k


def kernel(x, table):
    flat = x.reshape(x.shape[0] * x.shape[1])
    table2 = jnp.concatenate([table, table], axis=1)
    return _embed_lookup(x.shape[0], x.shape[1])(flat, table2)
